# Initial kernel scaffold; baseline (speedup 1.0000x reference)
import jax
import jax.numpy as jnp
from jax.experimental import pallas as pl


def kernel(x, Wg, bg, W1, b1, W2, b2):
    raise NotImplementedError("write your pallas kernel here")


# speedup vs baseline: 2.7900x; 2.7900x over previous
import functools

import jax
import jax.numpy as jnp
from jax import lax
from jax.experimental import pallas as pl
from jax.experimental.pallas import tpu as pltpu

_LANES = 128


def _moe_body(x32_ref, x16_ref, wg_ref, bg_ref, w1_ref, b1_ref, w2_ref,
              b2_ref, out_ref, gs_ref, *, n_real_experts):
    e = pl.program_id(1)

    @pl.when(e == 0)
    def _gating():
        l = jnp.dot(x32_ref[...], wg_ref[...],
                    preferred_element_type=jnp.float32) + bg_ref[...]
        col = lax.broadcasted_iota(jnp.int32, l.shape, 1)
        m1 = jnp.max(l, axis=1, keepdims=True)
        i1 = jnp.min(jnp.where(l == m1, col, _LANES), axis=1, keepdims=True)
        l2 = jnp.where(col == i1, -1e30, l)
        m2 = jnp.max(l2, axis=1, keepdims=True)
        i2 = jnp.min(jnp.where(l2 == m2, col, _LANES), axis=1, keepdims=True)
        z = jnp.exp(l - m1)
        p = z / jnp.sum(z, axis=1, keepdims=True)
        pm = jnp.where((col == i1) | (col == i2), p, 0.0)
        denom = jnp.maximum(jnp.sum(pm, axis=1, keepdims=True), 1e-12)
        gs_ref[...] = pm / denom
        out_ref[...] = jnp.zeros_like(out_ref)

    h = jnp.dot(x16_ref[...], w1_ref[0],
                preferred_element_type=jnp.float32) + b1_ref[0]
    h = 0.5 * h * (1.0 + lax.erf(h * 0.7071067811865476))
    y = jnp.dot(h.astype(jnp.bfloat16), w2_ref[0],
                preferred_element_type=jnp.float32) + b2_ref[0]
    col = lax.broadcasted_iota(jnp.int32, gs_ref.shape, 1)
    ge = jnp.sum(jnp.where(col == e, gs_ref[...], 0.0), axis=1, keepdims=True)
    out_ref[...] += ge * y


def kernel(x, Wg, bg, W1, b1, W2, b2):
    B, T, D = x.shape
    E = Wg.shape[1]
    H = W1.shape[2]
    N = B * T
    BLK = 512
    nt = N // BLK

    x32 = x.reshape(N, D)
    x16 = x32.astype(jnp.bfloat16)
    wg_pad = jnp.pad(Wg, ((0, 0), (0, _LANES - E)))
    bg_pad = jnp.pad(bg.reshape(1, E), ((0, 0), (0, _LANES - E)),
                     constant_values=-1e30)
    w1_16 = W1.astype(jnp.bfloat16)
    w2_16 = W2.astype(jnp.bfloat16)
    b1_3d = b1.reshape(E, 1, H)
    b2_3d = b2.reshape(E, 1, D)

    out = pl.pallas_call(
        functools.partial(_moe_body, n_real_experts=E),
        grid=(nt, E),
        in_specs=[
            pl.BlockSpec((BLK, D), lambda i, e: (i, 0)),
            pl.BlockSpec((BLK, D), lambda i, e: (i, 0)),
            pl.BlockSpec((D, _LANES), lambda i, e: (0, 0)),
            pl.BlockSpec((1, _LANES), lambda i, e: (0, 0)),
            pl.BlockSpec((1, D, H), lambda i, e: (e, 0, 0)),
            pl.BlockSpec((1, 1, H), lambda i, e: (e, 0, 0)),
            pl.BlockSpec((1, H, D), lambda i, e: (e, 0, 0)),
            pl.BlockSpec((1, 1, D), lambda i, e: (e, 0, 0)),
        ],
        out_specs=pl.BlockSpec((BLK, D), lambda i, e: (i, 0)),
        out_shape=jax.ShapeDtypeStruct((N, D), jnp.float32),
        scratch_shapes=[pltpu.VMEM((BLK, _LANES), jnp.float32)],
        compiler_params=pltpu.CompilerParams(
            dimension_semantics=("arbitrary", "arbitrary")),
    )(x32, x16, wg_pad, bg_pad, w1_16, b1_3d, w2_16, b2_3d)
    return out.reshape(B, T, D)


# speedup vs baseline: 3.5851x; 1.2850x over previous
import functools

import jax
import jax.numpy as jnp
from jax import lax
from jax.experimental import pallas as pl
from jax.experimental.pallas import tpu as pltpu
from jax.experimental.pallas import tpu_sc as plsc

_LANES = 128
_BLK = 256
_NB = 40
_SPAD = _NB * _BLK
_NW = 32
_SEG = 128


def _gating_body(x_ref, wg_ref, bg_ref, e1_ref, e2_ref, g1_ref, g2_ref,
                 cumh_ref, gp_ref, be_ref, br_ref, carry_ref, *, nblocks):
    b = pl.program_id(0)

    @pl.when(b == 0)
    def _():
        carry_ref[...] = jnp.zeros_like(carry_ref)

    l = jnp.dot(x_ref[...], wg_ref[...],
                preferred_element_type=jnp.float32) + bg_ref[...]
    bt = l.shape[0]
    col = lax.broadcasted_iota(jnp.int32, l.shape, 1)
    m1 = jnp.max(l, axis=1, keepdims=True)
    i1 = jnp.min(jnp.where(l == m1, col, _LANES), axis=1, keepdims=True)
    l2 = jnp.where(col == i1, -1e30, l)
    m2 = jnp.max(l2, axis=1, keepdims=True)
    i2 = jnp.min(jnp.where(l2 == m2, col, _LANES), axis=1, keepdims=True)
    z = jnp.exp(l - m1)
    p = z / jnp.sum(z, axis=1, keepdims=True)
    p1 = jnp.sum(jnp.where(col == i1, p, 0.0), axis=1, keepdims=True)
    p2 = jnp.sum(jnp.where(col == i2, p, 0.0), axis=1, keepdims=True)
    denom = jnp.maximum(p1 + p2, 1e-12)
    e1_ref[...] = i1.reshape(e1_ref.shape)
    e2_ref[...] = i2.reshape(e2_ref.shape)
    g1_ref[...] = jnp.broadcast_to(p1 / denom, g1_ref.shape)
    g2_ref[...] = jnp.broadcast_to(p2 / denom, g2_ref.shape)

    col16 = lax.broadcasted_iota(jnp.int32, (bt, 16), 1)
    cnts = ((col16 == i1).astype(jnp.float32)
            + (col16 == i2).astype(jnp.float32))
    seg = jnp.sum(cnts.reshape(bt // _SEG, _SEG, 16), axis=1)
    nseg = bt // _SEG
    r = lax.broadcasted_iota(jnp.int32, (nseg, nseg), 0)
    c = lax.broadcasted_iota(jnp.int32, (nseg, nseg), 1)
    strict_lower = (r > c).astype(jnp.float32)
    cum = jnp.dot(strict_lower, seg,
                  preferred_element_type=jnp.float32) + carry_ref[...]
    cumh_ref[...] = cum.astype(jnp.int32).reshape(cumh_ref.shape)
    carry_ref[...] = carry_ref[...] + jnp.sum(seg, axis=0, keepdims=True)

    @pl.when(b == nblocks - 1)
    def _():
        totals = carry_ref[...]
        pc = jnp.ceil(totals / _BLK) * _BLK
        ri = lax.broadcasted_iota(jnp.int32, (16, 16), 0)
        ci = lax.broadcasted_iota(jnp.int32, (16, 16), 1)
        u = ((ri < ci) & (ri < 8)).astype(jnp.float32)
        gp = jnp.dot(pc, u, preferred_element_type=jnp.float32)
        gp_ref[...] = gp.astype(jnp.int32)
        lane16 = lax.broadcasted_iota(jnp.int32, (1, 16), 1)
        lane128 = lax.broadcasted_iota(
            jnp.int32, (1, _LANES), 1).astype(jnp.float32)
        acc = jnp.full((1, _LANES), -1, jnp.int32)
        for e in range(8):
            ge = jnp.sum(jnp.where(lane16 == e, gp, 0.0))
            acc = acc + (lane128 * _BLK >= ge).astype(jnp.int32)
        be_ref[...] = jnp.clip(acc, 0, 7)
        total_pad = jnp.sum(jnp.where(lane16 == 8, gp, 0.0))
        br_ref[...] = (lane128 * _BLK < total_pad).astype(jnp.int32)


def _run_gating(x32, wg_pad, bg_pad, N, D):
    BT = 1024
    nblocks = N // BT
    return pl.pallas_call(
        functools.partial(_gating_body, nblocks=nblocks),
        grid=(nblocks,),
        in_specs=[
            pl.BlockSpec((BT, D), lambda b: (b, 0)),
            pl.BlockSpec((D, _LANES), lambda b: (0, 0)),
            pl.BlockSpec((1, _LANES), lambda b: (0, 0)),
        ],
        out_specs=[
            pl.BlockSpec((BT // _SEG, _SEG), lambda b: (b, 0)),
            pl.BlockSpec((BT // _SEG, _SEG), lambda b: (b, 0)),
            pl.BlockSpec((BT, 16), lambda b: (b, 0)),
            pl.BlockSpec((BT, 16), lambda b: (b, 0)),
            pl.BlockSpec((BT // _SEG, 1, 16), lambda b: (b, 0, 0)),
            pl.BlockSpec((1, 16), lambda b: (0, 0)),
            pl.BlockSpec((1, _LANES), lambda b: (0, 0)),
            pl.BlockSpec((1, _LANES), lambda b: (0, 0)),
        ],
        out_shape=[
            jax.ShapeDtypeStruct((N // _SEG, _SEG), jnp.int32),
            jax.ShapeDtypeStruct((N // _SEG, _SEG), jnp.int32),
            jax.ShapeDtypeStruct((N, 16), jnp.float32),
            jax.ShapeDtypeStruct((N, 16), jnp.float32),
            jax.ShapeDtypeStruct((N // _SEG, 1, 16), jnp.int32),
            jax.ShapeDtypeStruct((1, 16), jnp.int32),
            jax.ShapeDtypeStruct((1, _LANES), jnp.int32),
            jax.ShapeDtypeStruct((1, _LANES), jnp.int32),
        ],
        scratch_shapes=[pltpu.VMEM((1, 16), jnp.float32)],
        compiler_params=pltpu.CompilerParams(
            dimension_semantics=("arbitrary",)),
    )(x32, wg_pad, bg_pad)


def _dispatch_body(e1_hbm, e2_hbm, x_hbm, cumh_hbm, gp_hbm,
                   xs_hbm, d1_hbm, d2_hbm,
                   ev1, ev2, bc, gpv, base_r, cnt_r, d1_r, d2_r,
                   idx_r, xbuf, sem):
    nc = 2
    wid = lax.axis_index("s") * nc + lax.axis_index("c")
    tok = wid * _SEG
    pltpu.sync_copy(e1_hbm.at[pl.ds(tok, _SEG)], ev1)
    pltpu.sync_copy(e2_hbm.at[pl.ds(tok, _SEG)], ev2)
    pltpu.sync_copy(cumh_hbm.at[wid], bc)
    pltpu.sync_copy(gp_hbm, gpv)
    base_r[...] = bc[...] + gpv[...]
    cnt_r[...] = jnp.zeros((16,), jnp.int32)
    lane = lax.iota(jnp.int32, 16)
    for slot, (ev, d_r) in enumerate(((ev1, d1_r), (ev2, d2_r))):
        for v in range(_SEG // 16):
            eid = ev[pl.ds(v * 16, 16)]
            cntg = plsc.load_gather(cnt_r, [eid])
            baseg = plsc.load_gather(base_r, [eid])
            sp = jnp.zeros((16,), jnp.int32)
            hv = jnp.zeros((16,), jnp.int32)
            for e in range(8):
                m = (eid == e).astype(jnp.int32)
                cs = plsc.cumsum(m)
                sp = sp + (cs - m) * m
                hv = jnp.where(lane == e, jnp.sum(m), hv)
            d_r[pl.ds(v * 16, 16)] = baseg + cntg + sp
            cnt_r[...] = cnt_r[...] + hv
    pltpu.sync_copy(d1_r, d1_hbm.at[pl.ds(tok, _SEG)])
    pltpu.sync_copy(d2_r, d2_hbm.at[pl.ds(tok, _SEG)])
    for c in range(2):
        pltpu.sync_copy(x_hbm.at[pl.ds(tok + c * 64, 64)], xbuf)
        for d_r in (d1_r, d2_r):
            for q in range(4):
                idx_r[pl.ds(q * 16, 16)] = d_r[pl.ds(c * 64 + q * 16, 16)]
            pltpu.async_copy(xbuf, xs_hbm.at[idx_r], sem).wait()


def _run_dispatch(e1, e2, x32, cumh, gp, N, D):
    mesh = plsc.VectorSubcoreMesh(core_axis_name="c", subcore_axis_name="s")
    fn = pl.kernel(
        _dispatch_body,
        out_type=[
            jax.ShapeDtypeStruct((_SPAD, D), jnp.float32),
            jax.ShapeDtypeStruct((N,), jnp.int32),
            jax.ShapeDtypeStruct((N,), jnp.int32),
        ],
        mesh=mesh,
        scratch_types=[
            pltpu.VMEM((_SEG,), jnp.int32),
            pltpu.VMEM((_SEG,), jnp.int32),
            pltpu.VMEM((16,), jnp.int32),
            pltpu.VMEM((16,), jnp.int32),
            pltpu.VMEM((16,), jnp.int32),
            pltpu.VMEM((16,), jnp.int32),
            pltpu.VMEM((_SEG,), jnp.int32),
            pltpu.VMEM((_SEG,), jnp.int32),
            pltpu.VMEM((64,), jnp.int32),
            pltpu.VMEM((64, D), jnp.float32),
            pltpu.SemaphoreType.DMA,
        ],
        compiler_params=pltpu.CompilerParams(needs_layout_passes=False),
    )
    return fn(e1, e2, x32, cumh, gp)


def _ffn_body(be_ref, br_ref, xs_ref, w1_ref, b1_ref, w2_ref, b2_ref,
              ys_ref):
    b = pl.program_id(0)

    @pl.when(br_ref[b] == 1)
    def _():
        h = jnp.dot(xs_ref[...].astype(jnp.bfloat16), w1_ref[0],
                    preferred_element_type=jnp.float32) + b1_ref[0]
        h = 0.5 * h * (1.0 + lax.erf(h * 0.7071067811865476))
        ys_ref[...] = jnp.dot(h.astype(jnp.bfloat16), w2_ref[0],
                              preferred_element_type=jnp.float32) + b2_ref[0]


def _run_ffn(xs, be, br, w1_16, b1_3d, w2_16, b2_3d, D, H):
    grid_spec = pltpu.PrefetchScalarGridSpec(
        num_scalar_prefetch=2,
        grid=(_NB,),
        in_specs=[
            pl.BlockSpec((_BLK, D), lambda b, be, br: (b, 0)),
            pl.BlockSpec((1, D, H), lambda b, be, br: (be[b], 0, 0)),
            pl.BlockSpec((1, 1, H), lambda b, be, br: (be[b], 0, 0)),
            pl.BlockSpec((1, H, D), lambda b, be, br: (be[b], 0, 0)),
            pl.BlockSpec((1, 1, D), lambda b, be, br: (be[b], 0, 0)),
        ],
        out_specs=pl.BlockSpec((_BLK, D), lambda b, be, br: (b, 0)),
    )
    return pl.pallas_call(
        _ffn_body,
        grid_spec=grid_spec,
        out_shape=jax.ShapeDtypeStruct((_SPAD, D), jnp.float32),
        compiler_params=pltpu.CompilerParams(
            dimension_semantics=("arbitrary",)),
    )(be, br, xs, w1_16, b1_3d, w2_16, b2_3d)


def _combine_body(ys_hbm, d1_hbm, d2_hbm, g1_hbm, g2_hbm, out_hbm,
                  db1, db2, gb1, gb2, idx_r, yb1, yb2, sem):
    nc = 2
    wid = lax.axis_index("s") * nc + lax.axis_index("c")
    tok = wid * _SEG
    pltpu.sync_copy(d1_hbm.at[pl.ds(tok, _SEG)], db1)
    pltpu.sync_copy(d2_hbm.at[pl.ds(tok, _SEG)], db2)
    pltpu.sync_copy(g1_hbm.at[pl.ds(tok, _SEG)], gb1)
    pltpu.sync_copy(g2_hbm.at[pl.ds(tok, _SEG)], gb2)
    D = yb1.shape[1]
    for c in range(4):
        for q in range(2):
            idx_r[pl.ds(q * 16, 16)] = db1[pl.ds(c * 32 + q * 16, 16)]
        pltpu.async_copy(ys_hbm.at[idx_r], yb1, sem).wait()
        for q in range(2):
            idx_r[pl.ds(q * 16, 16)] = db2[pl.ds(c * 32 + q * 16, 16)]
        pltpu.async_copy(ys_hbm.at[idx_r], yb2, sem).wait()

        def row(j, _):
            g1v = gb1[c * 32 + j]
            g2v = gb2[c * 32 + j]

            def colk(k, __):
                a = (yb1[j, pl.ds(k * 16, 16)] * g1v
                     + yb2[j, pl.ds(k * 16, 16)] * g2v)
                yb1[j, pl.ds(k * 16, 16)] = a
                return __

            return lax.fori_loop(0, D // 16, colk, _)

        lax.fori_loop(0, 32, row, 0)
        pltpu.sync_copy(yb1, out_hbm.at[pl.ds(tok + c * 32, 32)])


def _run_combine(ys, d1, d2, g1, g2, N, D):
    mesh = plsc.VectorSubcoreMesh(core_axis_name="c", subcore_axis_name="s")
    fn = pl.kernel(
        _combine_body,
        out_type=jax.ShapeDtypeStruct((N, D), jnp.float32),
        mesh=mesh,
        scratch_types=[
            pltpu.VMEM((_SEG,), jnp.int32),
            pltpu.VMEM((_SEG,), jnp.int32),
            pltpu.VMEM((_SEG, 16), jnp.float32),
            pltpu.VMEM((_SEG, 16), jnp.float32),
            pltpu.VMEM((32,), jnp.int32),
            pltpu.VMEM((32, D), jnp.float32),
            pltpu.VMEM((32, D), jnp.float32),
            pltpu.SemaphoreType.DMA,
        ],
        compiler_params=pltpu.CompilerParams(needs_layout_passes=False),
    )
    return fn(ys, d1, d2, g1, g2)


def kernel(x, Wg, bg, W1, b1, W2, b2):
    B, T, D = x.shape
    E = Wg.shape[1]
    H = W1.shape[2]
    N = B * T

    x32 = x.reshape(N, D)
    wg_pad = jnp.pad(Wg, ((0, 0), (0, _LANES - E)))
    bg_pad = jnp.pad(bg.reshape(1, E), ((0, 0), (0, _LANES - E)),
                     constant_values=-1e30)
    w1_16 = W1.astype(jnp.bfloat16)
    w2_16 = W2.astype(jnp.bfloat16)
    b1_3d = b1.reshape(E, 1, H)
    b2_3d = b2.reshape(E, 1, D)

    e1, e2, g1b, g2b, cumh, gp, be128, br128 = _run_gating(
        x32, wg_pad, bg_pad, N, D)
    e1f = e1.reshape(N)
    e2f = e2.reshape(N)
    cumh2 = cumh.reshape(N // _SEG, 16)
    gpf = gp.reshape(16)
    be = be128.reshape(_LANES)[:_NB]
    br = br128.reshape(_LANES)[:_NB]

    xs, d1, d2 = _run_dispatch(e1f, e2f, x32, cumh2, gpf, N, D)
    ys = _run_ffn(xs, be, br, w1_16, b1_3d, w2_16, b2_3d, D, H)
    out = _run_combine(ys, d1, d2, g1b, g2b, N, D)
    return out.reshape(B, T, D)
